# baseline (speedup 1.0000x reference)
import functools

import jax
import jax.numpy as jnp
from jax import lax
from jax.experimental import pallas as pl
from jax.experimental.pallas import tpu as pltpu
from jax.experimental.pallas import tpu_sc as plsc


def _lane_iota():
    return lax.broadcasted_iota(jnp.int32, (16,), 0)


def _make_sc_src_gather(b, s, d, sp):
    info = plsc.get_sparse_core_info()
    nc, ns = info.num_cores, info.num_subcores
    nw = nc * ns
    assert b % nw == 0 and sp % 16 == 0
    nb = b // nw

    mesh = plsc.VectorSubcoreMesh(core_axis_name="c", subcore_axis_name="s")

    @functools.partial(
        pl.kernel,
        mesh=mesh,
        compiler_params=pltpu.CompilerParams(
            use_tc_tiling_on_sc=True, needs_layout_passes=False
        ),
        out_type=jax.ShapeDtypeStruct((b, s, d), jnp.float32),
        scratch_types=[
            pltpu.VMEM((sp,), jnp.int32),
            pltpu.VMEM((sp,), jnp.int32),
            pltpu.VMEM((sp, 2 * d), jnp.float32),
            pltpu.VMEM((s, d), jnp.float32),
            pltpu.SemaphoreType.DMA,
        ],
    )
    def src_gather(table2, idx_hbm, out_hbm, idx_v, ti_v, rows_v, ob_v, sem):
        wid = lax.axis_index("s") * nc + lax.axis_index("c")

        def batch_body(j, carry):
            bidx = wid * nb + j
            pltpu.sync_copy(idx_hbm.at[bidx], idx_v)
            for g in range(sp // 16):
                v = idx_v[pl.ds(g * 16, 16)]
                ti_v[pl.ds(g * 16, 16)] = lax.shift_right_logical(v, 1)
            pltpu.async_copy(table2.at[ti_v], rows_v, sem).wait()
            for g in range(sp // 16):
                n_valid = min(16, s - g * 16)
                if n_valid <= 0:
                    continue
                mask = (_lane_iota() < n_valid) if n_valid < 16 else None
                tok16 = _lane_iota() + g * 16
                half16 = lax.bitwise_and(idx_v[pl.ds(g * 16, 16)], 1) * d
                for c in range(d):
                    col16 = half16 + c
                    vals = plsc.load_gather(rows_v, [tok16, col16], mask=mask)
                    plsc.store_scatter(
                        ob_v, [tok16, jnp.full((16,), c, jnp.int32)], vals,
                        mask=mask,
                    )
            pltpu.sync_copy(ob_v, out_hbm.at[bidx])
            return carry

        lax.fori_loop(0, nb, batch_body, 0)

    return src_gather


def _make_tc_logits(b, s, v, d, n, block_b=8):
    assert b % block_b == 0

    def body(idx_ref, tgt_ref, log_ref, te_ref, ol_ref):
        for i in range(block_b):
            idxb = idx_ref[i]
            oh = (idxb == lax.broadcasted_iota(jnp.int32, (s, v), 1)).astype(
                jnp.bfloat16
            )
            te = jnp.dot(oh, tgt_ref[...], preferred_element_type=jnp.float32)
            te_ref[i] = te
            ol_ref[i] = jnp.dot(
                te.astype(jnp.bfloat16), log_ref[...],
                preferred_element_type=jnp.float32,
            )

    return pl.pallas_call(
        body,
        grid=(b // block_b,),
        in_specs=[
            pl.BlockSpec((block_b, s, 1), lambda i: (i, 0, 0)),
            pl.BlockSpec((v, d), lambda i: (0, 0)),
            pl.BlockSpec((d, n), lambda i: (0, 0)),
        ],
        out_specs=[
            pl.BlockSpec((block_b, s, d), lambda i: (i, 0, 0)),
            pl.BlockSpec((block_b, s, n), lambda i: (i, 0, 0)),
        ],
        out_shape=[
            jax.ShapeDtypeStruct((b, s, d), jnp.float32),
            jax.ShapeDtypeStruct((b, s, n), jnp.float32),
        ],
    )


def kernel(source_enumerate, target_enumerate, src_embs, tgt_embs, logits):
    b, s = source_enumerate.shape
    src_v, d = src_embs.shape
    tgt_v = tgt_embs.shape[0]
    n = logits.shape[1]

    sp = ((s + 15) // 16) * 16
    src_idx = jnp.pad(source_enumerate.astype(jnp.int32), ((0, 0), (0, sp - s)))
    table2 = src_embs.reshape(src_v // 2, 2 * d)
    src_emb = _make_sc_src_gather(b, s, d, sp)(table2, src_idx)

    idx3 = target_enumerate.reshape(b, s, 1).astype(jnp.int32)
    tgt_emb, out_logits = _make_tc_logits(b, s, tgt_v, d, n)(
        idx3, tgt_embs.astype(jnp.bfloat16), logits.astype(jnp.bfloat16)
    )
    return (src_emb, tgt_emb, out_logits)
